# baseline (speedup 1.0000x reference)
import functools

import jax
import jax.numpy as jnp
from jax import lax
from jax.experimental import pallas as pl
from jax.experimental.pallas import tpu as pltpu
from jax.experimental.pallas import tpu_sc as plsc

NC, NS, LANES = 2, 16, 16
NW = NC * NS
D = 128

N0, N1, N2 = 10000, 5000, 2500


def _mesh():
    return plsc.VectorSubcoreMesh(
        core_axis_name="c", subcore_axis_name="s", num_cores=NC, num_subcores=NS
    )


def _zero_fill(zeros_v):
    z16 = jnp.zeros((LANES,), jnp.float32)

    def zb(i, carry):
        for k in range(8):
            zeros_v[i, pl.ds(k * 16, 16)] = z16
        return carry

    lax.fori_loop(0, zeros_v.shape[0], zb, 0)


def _zero_acc(acc_sh, zeros_v, r0, nchunks):

    def zs(j, carry):
        pltpu.sync_copy(zeros_v, acc_sh.at[pl.ds(r0 + j * 32, 32), :])
        return carry

    lax.fori_loop(0, nchunks, zs, 0)


def _copy_out(acc_sh, out_hbm, c, r0, nchunks):

    def co(j, carry):
        sl = pl.ds(r0 + j * 32, 32)
        pltpu.sync_copy(acc_sh.at[sl, :], out_hbm.at[c, sl, :])
        return carry

    lax.fori_loop(0, nchunks, co, 0)


@functools.lru_cache(maxsize=None)
def _sc_histogram(np_rows, ep):
    epw = ep // NW
    nbt = epw // 128
    assert epw % 128 == 0 and np_rows % 16 == 0

    @functools.partial(
        pl.kernel,
        mesh=_mesh(),
        compiler_params=pltpu.CompilerParams(needs_layout_passes=False),
        out_type=jax.ShapeDtypeStruct((NW, np_rows), jnp.float32),
        scratch_types=[
            pltpu.VMEM((np_rows,), jnp.float32),
            pltpu.VMEM((128,), jnp.int32),
            pltpu.VMEM((128,), jnp.int32),
            pltpu.SemaphoreType.DMA,
        ],
    )
    def hist(dst_hbm, out_hbm, deg_v, da, db, isem):
        c = lax.axis_index("c")
        s = lax.axis_index("s")
        wid = s * NC + c
        base = wid * epw
        last = base + (nbt - 1) * 128

        def start(t, dv):
            off = jnp.minimum(base + t * 128, last)
            pltpu.async_copy(dst_hbm.at[pl.ds(off, 128)], dv, isem)

        def wait(dv):
            pltpu.make_async_copy(dst_hbm.at[pl.ds(0, 128)], dv, isem).wait()

        start(0, da)

        z16 = jnp.zeros((LANES,), jnp.float32)
        ones16 = jnp.ones((LANES,), jnp.float32)

        def zbody(i, carry):
            deg_v[pl.ds(i * 16, 16)] = z16
            return carry

        lax.fori_loop(0, np_rows // 16, zbody, 0)

        def step(t, dv, dvn):
            wait(dv)
            start(t + 1, dvn)
            for k in range(8):
                plsc.addupdate_scatter(deg_v, [dv[pl.ds(k * 16, 16)]], ones16)

        step(0, da, db)

        def ebody(m, carry):
            step(1 + 2 * m, db, da)
            step(2 + 2 * m, da, db)
            return carry

        lax.fori_loop(0, (nbt - 1) // 2, ebody, 0)
        if (nbt - 1) % 2:
            step(nbt - 1, db, da)
            wait(da)
        else:
            wait(db)
        pltpu.sync_copy(deg_v, out_hbm.at[wid])

    return hist


@functools.lru_cache(maxsize=None)
def _sc_propagate(ep, nd_pad):
    epw = ep // NW
    nbt = epw // 128
    ndw = nd_pad // NS
    assert epw % 128 == 0 and ndw % 32 == 0
    nzc = ndw // 32

    @functools.partial(
        pl.kernel,
        mesh=_mesh(),
        compiler_params=pltpu.CompilerParams(needs_layout_passes=False),
        out_type=jax.ShapeDtypeStruct((NC, nd_pad, D), jnp.float32),
        scratch_types=[
            pltpu.VMEM((128, D), jnp.float32),
            pltpu.VMEM((128,), jnp.int32),
            pltpu.VMEM((128,), jnp.int32),
            pltpu.VMEM((128,), jnp.int32),
            pltpu.VMEM((32, D), jnp.float32),
            pltpu.VMEM_SHARED((nd_pad, D), jnp.float32),
            pltpu.SemaphoreType.DMA,
            pltpu.SemaphoreType.DMA,
        ],
    )
    def prop(hs_hbm, src_hbm, dst_hbm, out_hbm,
             buf0, sa, da, db, zeros_v, acc_sh, gsem, ssem):
        c = lax.axis_index("c")
        s = lax.axis_index("s")
        wid = s * NC + c
        base = wid * epw

        pltpu.sync_copy(src_hbm.at[pl.ds(base, 128)], sa)
        pltpu.sync_copy(dst_hbm.at[pl.ds(base, 128)], da)
        pltpu.async_copy(hs_hbm.at[sa], buf0, gsem)

        _zero_fill(zeros_v)
        r0 = s * ndw
        _zero_acc(acc_sh, zeros_v, r0, nzc)
        plsc.subcore_barrier()

        def wait_s():
            pltpu.make_async_copy(buf0, acc_sh.at[da], ssem).wait()

        def step(t, dv, first):
            pltpu.sync_copy(src_hbm.at[pl.ds(base + t * 128, 128)], sa)
            pltpu.sync_copy(dst_hbm.at[pl.ds(base + t * 128, 128)], dv)
            if not first:
                wait_s()
            pltpu.async_copy(hs_hbm.at[sa], buf0, gsem).wait()
            pltpu.async_copy(buf0, acc_sh.at[dv], ssem, add=True)

        pltpu.make_async_copy(hs_hbm.at[sa], buf0, gsem).wait()
        pltpu.async_copy(buf0, acc_sh.at[da], ssem, add=True)

        def body(m, carry):
            step(1 + 2 * m, db, False)
            step(2 + 2 * m, da, False)
            return carry

        lax.fori_loop(0, (nbt - 1) // 2, body, 0)
        if (nbt - 1) % 2:
            step(nbt - 1, db, False)
        wait_s()
        plsc.subcore_barrier()
        _copy_out(acc_sh, out_hbm, c, r0, nzc)

    return prop


@functools.lru_cache(maxsize=None)
def _sc_pool(ns_pad, nd_pad):
    rpw = ns_pad // NW
    nb = rpw // 32
    ndw = nd_pad // NS
    assert rpw % 32 == 0 and ndw % 32 == 0

    @functools.partial(
        pl.kernel,
        mesh=_mesh(),
        compiler_params=pltpu.CompilerParams(needs_layout_passes=False),
        out_type=jax.ShapeDtypeStruct((NC, nd_pad, D), jnp.float32),
        scratch_types=[
            pltpu.VMEM((32, D), jnp.float32),
            pltpu.VMEM((32,), jnp.int32),
            pltpu.VMEM((32,), jnp.int32),
            pltpu.VMEM((32, D), jnp.float32),
            pltpu.VMEM_SHARED((nd_pad, D), jnp.float32),
            pltpu.SemaphoreType.DMA,
        ],
    )
    def pool(table_hbm, cl_hbm, out_hbm, rows_v, didx_v, didx2_v, zeros_v,
             acc_sh, ssem):
        c = lax.axis_index("c")
        s = lax.axis_index("s")
        wid = s * NC + c
        _zero_fill(zeros_v)
        r0 = s * ndw
        _zero_acc(acc_sh, zeros_v, r0, ndw // 32)
        plsc.subcore_barrier()

        base = wid * rpw

        def wait_s():
            pltpu.make_async_copy(rows_v, acc_sh.at[didx_v], ssem).wait()

        def step(j, dv, first):
            sl = pl.ds(base + j * 32, 32)
            pltpu.sync_copy(cl_hbm.at[sl], dv)
            if not first:
                wait_s()
            pltpu.sync_copy(table_hbm.at[sl, :], rows_v)
            pltpu.async_copy(rows_v, acc_sh.at[dv], ssem, add=True)

        step(0, didx_v, True)

        def eb(m, carry):
            step(1 + 2 * m, didx2_v, False)
            step(2 + 2 * m, didx_v, False)
            return carry

        lax.fori_loop(0, (nb - 1) // 2, eb, 0)
        if (nb - 1) % 2:
            step(nb - 1, didx2_v, False)
        wait_s()
        plsc.subcore_barrier()
        _copy_out(acc_sh, out_hbm, c, r0, ndw // 32)

    return pool


@functools.lru_cache(maxsize=None)
def _sc_gather(ng_pad):
    rpw = ng_pad // NW
    nb = rpw // 32
    assert rpw % 32 == 0

    @functools.partial(
        pl.kernel,
        mesh=_mesh(),
        compiler_params=pltpu.CompilerParams(needs_layout_passes=False),
        out_type=jax.ShapeDtypeStruct((ng_pad, D), jnp.float32),
        scratch_types=[
            pltpu.VMEM((32, D), jnp.float32),
            pltpu.VMEM((32,), jnp.int32),
            pltpu.SemaphoreType.DMA,
        ],
    )
    def gat(table_hbm, idx_hbm, out_hbm, rows_v, idx_v, sem):
        c = lax.axis_index("c")
        s = lax.axis_index("s")
        wid = s * NC + c
        base = wid * rpw

        def b(j, carry):
            sl = pl.ds(base + j * 32, 32)
            pltpu.sync_copy(idx_hbm.at[sl], idx_v)
            pltpu.async_copy(table_hbm.at[idx_v], rows_v, sem).wait()
            pltpu.sync_copy(rows_v, out_hbm.at[sl, :])
            return carry

        lax.fori_loop(0, nb, b, 0)

    return gat


_R = 256


@functools.lru_cache(maxsize=None)
def _tc_pre(n_pad, d_in, mode):
    grid = (n_pad // _R,)
    deg_spec = pl.BlockSpec((NW, _R, 1), lambda i: (0, i, 0))
    w_spec = pl.BlockSpec((d_in, D), lambda i: (0, 0))
    x2_spec = pl.BlockSpec((_R, d_in), lambda i: (i, 0))
    x3_spec = pl.BlockSpec((2, _R, d_in), lambda i: (0, i, 0))

    if mode == "one":
        in_specs = [x2_spec, deg_spec, w_spec]
    elif mode == "pair":
        in_specs = [x3_spec, deg_spec, w_spec]
    else:
        in_specs = [x2_spec, x2_spec, deg_spec, w_spec]

    def body(*refs):
        if mode == "one":
            x_ref, deg_ref, w_ref, o_ref = refs
            xb = x_ref[...]
        elif mode == "pair":
            x_ref, deg_ref, w_ref, o_ref = refs
            xb = x_ref[0] + x_ref[1]
        else:
            x_ref, s_ref, deg_ref, w_ref, o_ref = refs
            xb = x_ref[...] + s_ref[...]
        deg = jnp.sum(deg_ref[...], axis=0) + 1.0
        dis = lax.rsqrt(deg)
        h = jnp.dot(xb, w_ref[...], preferred_element_type=jnp.float32)
        o_ref[...] = h * dis

    return pl.pallas_call(
        body,
        grid=grid,
        in_specs=in_specs,
        out_specs=pl.BlockSpec((_R, D), lambda i: (i, 0)),
        out_shape=jax.ShapeDtypeStruct((n_pad, D), jnp.float32),
    )


@functools.lru_cache(maxsize=None)
def _tc_post(n_pad, final_rows):
    if final_rows:
        rb = 80
        n_out = final_rows
    else:
        rb = _R
        n_out = n_pad
    grid = (n_out // rb,)

    def body(p_ref, hs_ref, deg_ref, b_ref, o_ref):
        deg = jnp.sum(deg_ref[...], axis=0) + 1.0
        dis = lax.rsqrt(deg)
        t = (p_ref[0] + p_ref[1] + hs_ref[...]) * dis + b_ref[0:1, :]
        t = jnp.maximum(t, 0.0)
        if final_rows:
            nrm = jnp.sqrt(jnp.sum(t * t, axis=1, keepdims=True))
            t = t / jnp.maximum(nrm, 1e-12)
        o_ref[...] = t

    return pl.pallas_call(
        body,
        grid=grid,
        in_specs=[
            pl.BlockSpec((2, rb, D), lambda i: (0, i, 0)),
            pl.BlockSpec((rb, D), lambda i: (i, 0)),
            pl.BlockSpec((NW, rb, 1), lambda i: (0, i, 0)),
            pl.BlockSpec((8, D), lambda i: (0, 0)),
        ],
        out_specs=pl.BlockSpec((rb, D), lambda i: (i, 0)),
        out_shape=jax.ShapeDtypeStruct((n_out, D), jnp.float32),
    )


def _pad_idx(idx, n_pad, fill):
    return jnp.concatenate(
        [idx, jnp.full((n_pad - idx.shape[0],), fill, jnp.int32)]
    )


def kernel(x, edge_index, edge_index_l1, edge_index_l2, cluster0, cluster1,
           W_enc0, b_enc0, W_enc1, b_enc1, W_enc2, b_enc2, W_bot, b_bot,
           W_dec0, b_dec0, W_dec1, b_dec1, W_dec2, b_dec2):
    N0p, N1p, N2p = 10240, 5120, 2560
    E0p, E1p, E2p = 323584, 163840, 81920

    bcast = lambda b: jnp.broadcast_to(b.reshape(1, D), (8, D))

    s0 = _pad_idx(edge_index[0], E0p, N0)
    d0 = _pad_idx(edge_index[1], E0p, N0)
    s1 = _pad_idx(edge_index_l1[0], E1p, N1)
    d1 = _pad_idx(edge_index_l1[1], E1p, N1)
    s2 = _pad_idx(edge_index_l2[0], E2p, N2)
    d2 = _pad_idx(edge_index_l2[1], E2p, N2)
    s2b = _pad_idx(edge_index_l2[0], E2p, N1)
    d2b = _pad_idx(edge_index_l2[1], E2p, N1)
    s1b = _pad_idx(edge_index_l1[0], E1p, N0)
    d1b = _pad_idx(edge_index_l1[1], E1p, N0)
    c0p = _pad_idx(cluster0, N0p, N1)
    c1p = _pad_idx(cluster1, N1p, N2)
    c0g = _pad_idx(cluster0, N0p, 0)
    c1g = _pad_idx(cluster1, N1p, 0)
    xp = jnp.pad(x, ((0, N0p - N0), (0, 0)))

    hist0 = _sc_histogram(N0p, E0p)(d0)
    hist1 = _sc_histogram(N1p, E1p)(d1)
    hist2 = _sc_histogram(N2p, E2p)(d2)
    deg0 = hist0.reshape(NW, N0p, 1)
    deg1 = hist1.reshape(NW, N1p, 1)
    deg2 = hist2.reshape(NW, N2p, 1)
    deg2b = jnp.pad(hist2[:, :N2], ((0, 0), (0, N1p - N2))).reshape(NW, N1p, 1)
    deg1b = jnp.pad(hist1[:, :N1], ((0, 0), (0, N0p - N1))).reshape(NW, N0p, 1)

    hs0 = _tc_pre(N0p, D, "one")(xp, deg0, W_enc0)
    p0 = _sc_propagate(E0p, N0p)(hs0, s0, d0)
    out0 = _tc_post(N0p, 0)(p0, hs0, deg0, bcast(b_enc0))

    q1 = _sc_pool(N0p, N1p)(out0, c0p)
    hs1 = _tc_pre(N1p, D, "pair")(q1, deg1, W_enc1)
    p1 = _sc_propagate(E1p, N1p)(hs1, s1, d1)
    out1 = _tc_post(N1p, 0)(p1, hs1, deg1, bcast(b_enc1))

    q2 = _sc_pool(N1p, N2p)(out1, c1p)
    hs2 = _tc_pre(N2p, D, "pair")(q2, deg2, W_enc2)
    p2 = _sc_propagate(E2p, N2p)(hs2, s2, d2)
    out2 = _tc_post(N2p, 0)(p2, hs2, deg2, bcast(b_enc2))

    hsb = _tc_pre(N2p, D, "one")(out2, deg2, W_bot)
    pb = _sc_propagate(E2p, N2p)(hsb, s2, d2)
    outb = _tc_post(N2p, 0)(pb, hsb, deg2, bcast(b_bot))

    g1 = _sc_gather(N1p)(outb, c1g)
    hsd2 = _tc_pre(N1p, D, "two")(g1, out1, deg2b, W_dec2)
    pd2 = _sc_propagate(E2p, N1p)(hsd2, s2b, d2b)
    outd2 = _tc_post(N1p, 0)(pd2, hsd2, deg2b, bcast(b_dec2))

    g0 = _sc_gather(N0p)(outd2, c0g)
    hsd1 = _tc_pre(N0p, D, "two")(g0, out0, deg1b, W_dec1)
    pd1 = _sc_propagate(E1p, N0p)(hsd1, s1b, d1b)
    outd1 = _tc_post(N0p, 0)(pd1, hsd1, deg1b, bcast(b_dec1))

    hsd0 = _tc_pre(N0p, D, "one")(outd1, deg0, W_dec0)
    pd0 = _sc_propagate(E0p, N0p)(hsd0, s0, d0)
    return _tc_post(N0p, N0)(pd0, hsd0, deg0, bcast(b_dec0))


# speedup vs baseline: 1.0066x; 1.0066x over previous
import functools

import jax
import jax.numpy as jnp
from jax import lax
from jax.experimental import pallas as pl
from jax.experimental.pallas import tpu as pltpu
from jax.experimental.pallas import tpu_sc as plsc

NC, NS, LANES = 2, 16, 16
NW = NC * NS
D = 128

N0, N1, N2 = 10000, 5000, 2500


def _mesh():
    return plsc.VectorSubcoreMesh(
        core_axis_name="c", subcore_axis_name="s", num_cores=NC, num_subcores=NS
    )


def _zero_fill(zeros_v):
    z16 = jnp.zeros((LANES,), jnp.float32)

    def zb(i, carry):
        for k in range(8):
            zeros_v[i, pl.ds(k * 16, 16)] = z16
        return carry

    lax.fori_loop(0, zeros_v.shape[0], zb, 0)


def _zero_acc(acc_sh, zeros_v, r0, ndw):
    zc = zeros_v.shape[0]

    def zs(j, carry):
        pltpu.sync_copy(zeros_v, acc_sh.at[pl.ds(r0 + j * zc, zc), :])
        return carry

    lax.fori_loop(0, ndw // zc, zs, 0)


def _copy_out(acc_sh, out_hbm, c, r0, ndw):
    cc = ndw // 4
    for j in range(4):
        sl = pl.ds(r0 + j * cc, cc)
        pltpu.sync_copy(acc_sh.at[sl, :], out_hbm.at[c, sl, :])


@functools.lru_cache(maxsize=None)
def _sc_histogram(np_rows, ep):
    epw = ep // NW
    nbt = epw // 128
    assert epw % 128 == 0 and np_rows % 16 == 0

    @functools.partial(
        pl.kernel,
        mesh=_mesh(),
        compiler_params=pltpu.CompilerParams(needs_layout_passes=False),
        out_type=jax.ShapeDtypeStruct((NW, np_rows), jnp.float32),
        scratch_types=[
            pltpu.VMEM((np_rows,), jnp.float32),
            pltpu.VMEM((128,), jnp.int32),
            pltpu.VMEM((128,), jnp.int32),
            pltpu.SemaphoreType.DMA,
        ],
    )
    def hist(dst_hbm, out_hbm, deg_v, da, db, isem):
        c = lax.axis_index("c")
        s = lax.axis_index("s")
        wid = s * NC + c
        base = wid * epw
        last = base + (nbt - 1) * 128

        def start(t, dv):
            off = jnp.minimum(base + t * 128, last)
            pltpu.async_copy(dst_hbm.at[pl.ds(off, 128)], dv, isem)

        def wait(dv):
            pltpu.make_async_copy(dst_hbm.at[pl.ds(0, 128)], dv, isem).wait()

        start(0, da)

        z16 = jnp.zeros((LANES,), jnp.float32)
        ones16 = jnp.ones((LANES,), jnp.float32)

        def zbody(i, carry):
            deg_v[pl.ds(i * 16, 16)] = z16
            return carry

        lax.fori_loop(0, np_rows // 16, zbody, 0)

        def step(t, dv, dvn):
            wait(dv)
            start(t + 1, dvn)
            for k in range(8):
                plsc.addupdate_scatter(deg_v, [dv[pl.ds(k * 16, 16)]], ones16)

        step(0, da, db)

        def ebody(m, carry):
            step(1 + 2 * m, db, da)
            step(2 + 2 * m, da, db)
            return carry

        lax.fori_loop(0, (nbt - 1) // 2, ebody, 0)
        if (nbt - 1) % 2:
            step(nbt - 1, db, da)
            wait(da)
        else:
            wait(db)
        pltpu.sync_copy(deg_v, out_hbm.at[wid])

    return hist


@functools.lru_cache(maxsize=None)
def _sc_propagate(ep, nd_pad):
    epw = ep // NW
    nbt = epw // 128
    ndw = nd_pad // NS
    assert epw % 128 == 0 and ndw % 32 == 0
    zc = 64 if ndw % 64 == 0 else 32

    @functools.partial(
        pl.kernel,
        mesh=_mesh(),
        compiler_params=pltpu.CompilerParams(needs_layout_passes=False),
        out_type=jax.ShapeDtypeStruct((NC, nd_pad, D), jnp.float32),
        scratch_types=[
            pltpu.VMEM((128, D), jnp.float32),
            pltpu.VMEM((128,), jnp.int32),
            pltpu.VMEM((128,), jnp.int32),
            pltpu.VMEM((128,), jnp.int32),
            pltpu.VMEM((zc, D), jnp.float32),
            pltpu.VMEM_SHARED((nd_pad, D), jnp.float32),
            pltpu.SemaphoreType.DMA,
            pltpu.SemaphoreType.DMA,
        ],
    )
    def prop(hs_hbm, src_hbm, dst_hbm, out_hbm,
             buf0, sa, da, db, zeros_v, acc_sh, gsem, ssem):
        c = lax.axis_index("c")
        s = lax.axis_index("s")
        wid = s * NC + c
        base = wid * epw

        pltpu.sync_copy(src_hbm.at[pl.ds(base, 128)], sa)
        pltpu.sync_copy(dst_hbm.at[pl.ds(base, 128)], da)
        pltpu.async_copy(hs_hbm.at[sa], buf0, gsem)

        _zero_fill(zeros_v)
        r0 = s * ndw
        _zero_acc(acc_sh, zeros_v, r0, ndw)
        plsc.subcore_barrier()

        def wait_s():
            pltpu.make_async_copy(buf0, acc_sh.at[da], ssem).wait()

        def step(t, dv, first):
            pltpu.sync_copy(src_hbm.at[pl.ds(base + t * 128, 128)], sa)
            pltpu.sync_copy(dst_hbm.at[pl.ds(base + t * 128, 128)], dv)
            if not first:
                wait_s()
            pltpu.async_copy(hs_hbm.at[sa], buf0, gsem).wait()
            pltpu.async_copy(buf0, acc_sh.at[dv], ssem, add=True)

        pltpu.make_async_copy(hs_hbm.at[sa], buf0, gsem).wait()
        pltpu.async_copy(buf0, acc_sh.at[da], ssem, add=True)

        def body(m, carry):
            step(1 + 2 * m, db, False)
            step(2 + 2 * m, da, False)
            return carry

        lax.fori_loop(0, (nbt - 1) // 2, body, 0)
        if (nbt - 1) % 2:
            step(nbt - 1, db, False)
        wait_s()
        plsc.subcore_barrier()
        _copy_out(acc_sh, out_hbm, c, r0, ndw)

    return prop


@functools.lru_cache(maxsize=None)
def _sc_pool(ns_pad, nd_pad):
    rpw = ns_pad // NW
    nb = rpw // 32
    ndw = nd_pad // NS
    assert rpw % 32 == 0 and ndw % 32 == 0
    zc = 64 if ndw % 64 == 0 else 32

    @functools.partial(
        pl.kernel,
        mesh=_mesh(),
        compiler_params=pltpu.CompilerParams(needs_layout_passes=False),
        out_type=jax.ShapeDtypeStruct((NC, nd_pad, D), jnp.float32),
        scratch_types=[
            pltpu.VMEM((32, D), jnp.float32),
            pltpu.VMEM((32,), jnp.int32),
            pltpu.VMEM((32,), jnp.int32),
            pltpu.VMEM((zc, D), jnp.float32),
            pltpu.VMEM_SHARED((nd_pad, D), jnp.float32),
            pltpu.SemaphoreType.DMA,
        ],
    )
    def pool(table_hbm, cl_hbm, out_hbm, rows_v, didx_v, didx2_v, zeros_v,
             acc_sh, ssem):
        c = lax.axis_index("c")
        s = lax.axis_index("s")
        wid = s * NC + c
        _zero_fill(zeros_v)
        r0 = s * ndw
        _zero_acc(acc_sh, zeros_v, r0, ndw)
        plsc.subcore_barrier()

        base = wid * rpw

        def wait_s():
            pltpu.make_async_copy(rows_v, acc_sh.at[didx_v], ssem).wait()

        def step(j, dv, first):
            sl = pl.ds(base + j * 32, 32)
            pltpu.sync_copy(cl_hbm.at[sl], dv)
            if not first:
                wait_s()
            pltpu.sync_copy(table_hbm.at[sl, :], rows_v)
            pltpu.async_copy(rows_v, acc_sh.at[dv], ssem, add=True)

        step(0, didx_v, True)

        def eb(m, carry):
            step(1 + 2 * m, didx2_v, False)
            step(2 + 2 * m, didx_v, False)
            return carry

        lax.fori_loop(0, (nb - 1) // 2, eb, 0)
        if (nb - 1) % 2:
            step(nb - 1, didx2_v, False)
        wait_s()
        plsc.subcore_barrier()
        _copy_out(acc_sh, out_hbm, c, r0, ndw)

    return pool


@functools.lru_cache(maxsize=None)
def _sc_gather(ng_pad):
    rpw = ng_pad // NW
    nb = rpw // 32
    assert rpw % 32 == 0

    @functools.partial(
        pl.kernel,
        mesh=_mesh(),
        compiler_params=pltpu.CompilerParams(needs_layout_passes=False),
        out_type=jax.ShapeDtypeStruct((ng_pad, D), jnp.float32),
        scratch_types=[
            pltpu.VMEM((32, D), jnp.float32),
            pltpu.VMEM((32,), jnp.int32),
            pltpu.SemaphoreType.DMA,
        ],
    )
    def gat(table_hbm, idx_hbm, out_hbm, rows_v, idx_v, sem):
        c = lax.axis_index("c")
        s = lax.axis_index("s")
        wid = s * NC + c
        base = wid * rpw

        def b(j, carry):
            sl = pl.ds(base + j * 32, 32)
            pltpu.sync_copy(idx_hbm.at[sl], idx_v)
            pltpu.async_copy(table_hbm.at[idx_v], rows_v, sem).wait()
            pltpu.sync_copy(rows_v, out_hbm.at[sl, :])
            return carry

        lax.fori_loop(0, nb, b, 0)

    return gat


_R = 256


@functools.lru_cache(maxsize=None)
def _tc_pre(n_pad, d_in, mode):
    grid = (n_pad // _R,)
    deg_spec = pl.BlockSpec((NW, _R, 1), lambda i: (0, i, 0))
    w_spec = pl.BlockSpec((d_in, D), lambda i: (0, 0))
    x2_spec = pl.BlockSpec((_R, d_in), lambda i: (i, 0))
    x3_spec = pl.BlockSpec((2, _R, d_in), lambda i: (0, i, 0))

    if mode == "one":
        in_specs = [x2_spec, deg_spec, w_spec]
    elif mode == "pair":
        in_specs = [x3_spec, deg_spec, w_spec]
    else:
        in_specs = [x2_spec, x2_spec, deg_spec, w_spec]

    def body(*refs):
        if mode == "one":
            x_ref, deg_ref, w_ref, o_ref = refs
            xb = x_ref[...]
        elif mode == "pair":
            x_ref, deg_ref, w_ref, o_ref = refs
            xb = x_ref[0] + x_ref[1]
        else:
            x_ref, s_ref, deg_ref, w_ref, o_ref = refs
            xb = x_ref[...] + s_ref[...]
        deg = jnp.sum(deg_ref[...], axis=0) + 1.0
        dis = lax.rsqrt(deg)
        h = jnp.dot(xb, w_ref[...], preferred_element_type=jnp.float32)
        o_ref[...] = h * dis

    return pl.pallas_call(
        body,
        grid=grid,
        in_specs=in_specs,
        out_specs=pl.BlockSpec((_R, D), lambda i: (i, 0)),
        out_shape=jax.ShapeDtypeStruct((n_pad, D), jnp.float32),
    )


@functools.lru_cache(maxsize=None)
def _tc_post(n_pad, final_rows):
    if final_rows:
        rb = 80
        n_out = final_rows
    else:
        rb = _R
        n_out = n_pad
    grid = (n_out // rb,)

    def body(p_ref, hs_ref, deg_ref, b_ref, o_ref):
        deg = jnp.sum(deg_ref[...], axis=0) + 1.0
        dis = lax.rsqrt(deg)
        t = (p_ref[0] + p_ref[1] + hs_ref[...]) * dis + b_ref[0:1, :]
        t = jnp.maximum(t, 0.0)
        if final_rows:
            nrm = jnp.sqrt(jnp.sum(t * t, axis=1, keepdims=True))
            t = t / jnp.maximum(nrm, 1e-12)
        o_ref[...] = t

    return pl.pallas_call(
        body,
        grid=grid,
        in_specs=[
            pl.BlockSpec((2, rb, D), lambda i: (0, i, 0)),
            pl.BlockSpec((rb, D), lambda i: (i, 0)),
            pl.BlockSpec((NW, rb, 1), lambda i: (0, i, 0)),
            pl.BlockSpec((8, D), lambda i: (0, 0)),
        ],
        out_specs=pl.BlockSpec((rb, D), lambda i: (i, 0)),
        out_shape=jax.ShapeDtypeStruct((n_out, D), jnp.float32),
    )


def _pad_idx(idx, n_pad, fill):
    return jnp.concatenate(
        [idx, jnp.full((n_pad - idx.shape[0],), fill, jnp.int32)]
    )


def kernel(x, edge_index, edge_index_l1, edge_index_l2, cluster0, cluster1,
           W_enc0, b_enc0, W_enc1, b_enc1, W_enc2, b_enc2, W_bot, b_bot,
           W_dec0, b_dec0, W_dec1, b_dec1, W_dec2, b_dec2):
    N0p, N1p, N2p = 10240, 5120, 2560
    E0p, E1p, E2p = 323584, 163840, 81920

    bcast = lambda b: jnp.broadcast_to(b.reshape(1, D), (8, D))

    s0 = _pad_idx(edge_index[0], E0p, N0)
    d0 = _pad_idx(edge_index[1], E0p, N0)
    s1 = _pad_idx(edge_index_l1[0], E1p, N1)
    d1 = _pad_idx(edge_index_l1[1], E1p, N1)
    s2 = _pad_idx(edge_index_l2[0], E2p, N2)
    d2 = _pad_idx(edge_index_l2[1], E2p, N2)
    s2b = _pad_idx(edge_index_l2[0], E2p, N1)
    d2b = _pad_idx(edge_index_l2[1], E2p, N1)
    s1b = _pad_idx(edge_index_l1[0], E1p, N0)
    d1b = _pad_idx(edge_index_l1[1], E1p, N0)
    c0p = _pad_idx(cluster0, N0p, N1)
    c1p = _pad_idx(cluster1, N1p, N2)
    c0g = _pad_idx(cluster0, N0p, 0)
    c1g = _pad_idx(cluster1, N1p, 0)
    xp = jnp.pad(x, ((0, N0p - N0), (0, 0)))

    hist0 = _sc_histogram(N0p, E0p)(d0)
    hist1 = _sc_histogram(N1p, E1p)(d1)
    hist2 = _sc_histogram(N2p, E2p)(d2)
    deg0 = hist0.reshape(NW, N0p, 1)
    deg1 = hist1.reshape(NW, N1p, 1)
    deg2 = hist2.reshape(NW, N2p, 1)
    deg2b = jnp.pad(hist2[:, :N2], ((0, 0), (0, N1p - N2))).reshape(NW, N1p, 1)
    deg1b = jnp.pad(hist1[:, :N1], ((0, 0), (0, N0p - N1))).reshape(NW, N0p, 1)

    hs0 = _tc_pre(N0p, D, "one")(xp, deg0, W_enc0)
    p0 = _sc_propagate(E0p, N0p)(hs0, s0, d0)
    out0 = _tc_post(N0p, 0)(p0, hs0, deg0, bcast(b_enc0))

    q1 = _sc_pool(N0p, N1p)(out0, c0p)
    hs1 = _tc_pre(N1p, D, "pair")(q1, deg1, W_enc1)
    p1 = _sc_propagate(E1p, N1p)(hs1, s1, d1)
    out1 = _tc_post(N1p, 0)(p1, hs1, deg1, bcast(b_enc1))

    q2 = _sc_pool(N1p, N2p)(out1, c1p)
    hs2 = _tc_pre(N2p, D, "pair")(q2, deg2, W_enc2)
    p2 = _sc_propagate(E2p, N2p)(hs2, s2, d2)
    out2 = _tc_post(N2p, 0)(p2, hs2, deg2, bcast(b_enc2))

    hsb = _tc_pre(N2p, D, "one")(out2, deg2, W_bot)
    pb = _sc_propagate(E2p, N2p)(hsb, s2, d2)
    outb = _tc_post(N2p, 0)(pb, hsb, deg2, bcast(b_bot))

    g1 = _sc_gather(N1p)(outb, c1g)
    hsd2 = _tc_pre(N1p, D, "two")(g1, out1, deg2b, W_dec2)
    pd2 = _sc_propagate(E2p, N1p)(hsd2, s2b, d2b)
    outd2 = _tc_post(N1p, 0)(pd2, hsd2, deg2b, bcast(b_dec2))

    g0 = _sc_gather(N0p)(outd2, c0g)
    hsd1 = _tc_pre(N0p, D, "two")(g0, out0, deg1b, W_dec1)
    pd1 = _sc_propagate(E1p, N0p)(hsd1, s1b, d1b)
    outd1 = _tc_post(N0p, 0)(pd1, hsd1, deg1b, bcast(b_dec1))

    hsd0 = _tc_pre(N0p, D, "one")(outd1, deg0, W_dec0)
    pd0 = _sc_propagate(E0p, N0p)(hsd0, s0, d0)
    return _tc_post(N0p, N0)(pd0, hsd0, deg0, bcast(b_dec0))


# speedup vs baseline: 1.0101x; 1.0035x over previous
import functools

import jax
import jax.numpy as jnp
from jax import lax
from jax.experimental import pallas as pl
from jax.experimental.pallas import tpu as pltpu
from jax.experimental.pallas import tpu_sc as plsc

NC, NS, LANES = 2, 16, 16
NW = NC * NS
D = 128

N0, N1, N2 = 10000, 5000, 2500


def _mesh():
    return plsc.VectorSubcoreMesh(
        core_axis_name="c", subcore_axis_name="s", num_cores=NC, num_subcores=NS
    )


def _zero_fill(zeros_v):
    z16 = jnp.zeros((LANES,), jnp.float32)

    def zb(i, carry):
        for k in range(8):
            zeros_v[i, pl.ds(k * 16, 16)] = z16
        return carry

    lax.fori_loop(0, zeros_v.shape[0], zb, 0)


def _zero_acc(acc_sh, zeros_v, r0, ndw):
    zc = zeros_v.shape[0]

    def zs(j, carry):
        pltpu.sync_copy(zeros_v, acc_sh.at[pl.ds(r0 + j * zc, zc), :])
        return carry

    lax.fori_loop(0, ndw // zc, zs, 0)


def _copy_out(acc_sh, out_hbm, c, r0, ndw):
    cc = ndw // 2
    for j in range(2):
        sl = pl.ds(r0 + j * cc, cc)
        pltpu.sync_copy(acc_sh.at[sl, :], out_hbm.at[c, sl, :])


@functools.lru_cache(maxsize=None)
def _sc_histogram(np_rows, ep):
    epw = ep // NW
    nbt = epw // 128
    assert epw % 128 == 0 and np_rows % 16 == 0

    @functools.partial(
        pl.kernel,
        mesh=_mesh(),
        compiler_params=pltpu.CompilerParams(needs_layout_passes=False),
        out_type=jax.ShapeDtypeStruct((NW, np_rows), jnp.float32),
        scratch_types=[
            pltpu.VMEM((np_rows,), jnp.float32),
            pltpu.VMEM((128,), jnp.int32),
            pltpu.VMEM((128,), jnp.int32),
            pltpu.SemaphoreType.DMA,
        ],
    )
    def hist(dst_hbm, out_hbm, deg_v, da, db, isem):
        c = lax.axis_index("c")
        s = lax.axis_index("s")
        wid = s * NC + c
        base = wid * epw
        last = base + (nbt - 1) * 128

        def start(t, dv):
            off = jnp.minimum(base + t * 128, last)
            pltpu.async_copy(dst_hbm.at[pl.ds(off, 128)], dv, isem)

        def wait(dv):
            pltpu.make_async_copy(dst_hbm.at[pl.ds(0, 128)], dv, isem).wait()

        start(0, da)

        z16 = jnp.zeros((LANES,), jnp.float32)
        ones16 = jnp.ones((LANES,), jnp.float32)

        def zbody(i, carry):
            deg_v[pl.ds(i * 16, 16)] = z16
            return carry

        lax.fori_loop(0, np_rows // 16, zbody, 0)

        def step(t, dv, dvn):
            wait(dv)
            start(t + 1, dvn)
            for k in range(8):
                plsc.addupdate_scatter(deg_v, [dv[pl.ds(k * 16, 16)]], ones16)

        step(0, da, db)

        def ebody(m, carry):
            step(1 + 2 * m, db, da)
            step(2 + 2 * m, da, db)
            return carry

        lax.fori_loop(0, (nbt - 1) // 2, ebody, 0)
        if (nbt - 1) % 2:
            step(nbt - 1, db, da)
            wait(da)
        else:
            wait(db)
        pltpu.sync_copy(deg_v, out_hbm.at[wid])

    return hist


@functools.lru_cache(maxsize=None)
def _sc_propagate(ep, nd_pad):
    epw = ep // NW
    nbt = epw // 128
    ndw = nd_pad // NS
    assert epw % 128 == 0 and ndw % 32 == 0
    zc = 64 if ndw % 64 == 0 else 32

    @functools.partial(
        pl.kernel,
        mesh=_mesh(),
        compiler_params=pltpu.CompilerParams(needs_layout_passes=False),
        out_type=jax.ShapeDtypeStruct((NC, nd_pad, D), jnp.float32),
        scratch_types=[
            pltpu.VMEM((128, D), jnp.float32),
            pltpu.VMEM((128,), jnp.int32),
            pltpu.VMEM((128,), jnp.int32),
            pltpu.VMEM((128,), jnp.int32),
            pltpu.VMEM((zc, D), jnp.float32),
            pltpu.VMEM_SHARED((nd_pad, D), jnp.float32),
            pltpu.SemaphoreType.DMA,
            pltpu.SemaphoreType.DMA,
        ],
    )
    def prop(hs_hbm, src_hbm, dst_hbm, out_hbm,
             buf0, sa, da, db, zeros_v, acc_sh, gsem, ssem):
        c = lax.axis_index("c")
        s = lax.axis_index("s")
        wid = s * NC + c
        base = wid * epw

        pltpu.sync_copy(src_hbm.at[pl.ds(base, 128)], sa)
        pltpu.sync_copy(dst_hbm.at[pl.ds(base, 128)], da)
        pltpu.async_copy(hs_hbm.at[sa], buf0, gsem)

        _zero_fill(zeros_v)
        r0 = s * ndw
        _zero_acc(acc_sh, zeros_v, r0, ndw)
        plsc.subcore_barrier()

        def wait_s():
            pltpu.make_async_copy(buf0, acc_sh.at[da], ssem).wait()

        def step(t, dv, first):
            pltpu.sync_copy(src_hbm.at[pl.ds(base + t * 128, 128)], sa)
            pltpu.sync_copy(dst_hbm.at[pl.ds(base + t * 128, 128)], dv)
            if not first:
                wait_s()
            pltpu.async_copy(hs_hbm.at[sa], buf0, gsem).wait()
            pltpu.async_copy(buf0, acc_sh.at[dv], ssem, add=True)

        pltpu.make_async_copy(hs_hbm.at[sa], buf0, gsem).wait()
        pltpu.async_copy(buf0, acc_sh.at[da], ssem, add=True)

        def body(m, carry):
            step(1 + 2 * m, db, False)
            step(2 + 2 * m, da, False)
            return carry

        lax.fori_loop(0, (nbt - 1) // 2, body, 0)
        if (nbt - 1) % 2:
            step(nbt - 1, db, False)
        wait_s()
        plsc.subcore_barrier()
        _copy_out(acc_sh, out_hbm, c, r0, ndw)

    return prop


@functools.lru_cache(maxsize=None)
def _sc_pool(ns_pad, nd_pad):
    rpw = ns_pad // NW
    nb = rpw // 32
    ndw = nd_pad // NS
    assert rpw % 32 == 0 and ndw % 32 == 0
    zc = 64 if ndw % 64 == 0 else 32

    @functools.partial(
        pl.kernel,
        mesh=_mesh(),
        compiler_params=pltpu.CompilerParams(needs_layout_passes=False),
        out_type=jax.ShapeDtypeStruct((NC, nd_pad, D), jnp.float32),
        scratch_types=[
            pltpu.VMEM((32, D), jnp.float32),
            pltpu.VMEM((32,), jnp.int32),
            pltpu.VMEM((32,), jnp.int32),
            pltpu.VMEM((zc, D), jnp.float32),
            pltpu.VMEM_SHARED((nd_pad, D), jnp.float32),
            pltpu.SemaphoreType.DMA,
        ],
    )
    def pool(table_hbm, cl_hbm, out_hbm, rows_v, didx_v, didx2_v, zeros_v,
             acc_sh, ssem):
        c = lax.axis_index("c")
        s = lax.axis_index("s")
        wid = s * NC + c
        _zero_fill(zeros_v)
        r0 = s * ndw
        _zero_acc(acc_sh, zeros_v, r0, ndw)
        plsc.subcore_barrier()

        base = wid * rpw

        def wait_s():
            pltpu.make_async_copy(rows_v, acc_sh.at[didx_v], ssem).wait()

        def step(j, dv, first):
            sl = pl.ds(base + j * 32, 32)
            pltpu.sync_copy(cl_hbm.at[sl], dv)
            if not first:
                wait_s()
            pltpu.sync_copy(table_hbm.at[sl, :], rows_v)
            pltpu.async_copy(rows_v, acc_sh.at[dv], ssem, add=True)

        step(0, didx_v, True)

        def eb(m, carry):
            step(1 + 2 * m, didx2_v, False)
            step(2 + 2 * m, didx_v, False)
            return carry

        lax.fori_loop(0, (nb - 1) // 2, eb, 0)
        if (nb - 1) % 2:
            step(nb - 1, didx2_v, False)
        wait_s()
        plsc.subcore_barrier()
        _copy_out(acc_sh, out_hbm, c, r0, ndw)

    return pool


@functools.lru_cache(maxsize=None)
def _sc_gather(ng_pad):
    rpw = ng_pad // NW
    nb = rpw // 32
    assert rpw % 32 == 0

    @functools.partial(
        pl.kernel,
        mesh=_mesh(),
        compiler_params=pltpu.CompilerParams(needs_layout_passes=False),
        out_type=jax.ShapeDtypeStruct((ng_pad, D), jnp.float32),
        scratch_types=[
            pltpu.VMEM((32, D), jnp.float32),
            pltpu.VMEM((32,), jnp.int32),
            pltpu.SemaphoreType.DMA,
        ],
    )
    def gat(table_hbm, idx_hbm, out_hbm, rows_v, idx_v, sem):
        c = lax.axis_index("c")
        s = lax.axis_index("s")
        wid = s * NC + c
        base = wid * rpw

        def b(j, carry):
            sl = pl.ds(base + j * 32, 32)
            pltpu.sync_copy(idx_hbm.at[sl], idx_v)
            pltpu.async_copy(table_hbm.at[idx_v], rows_v, sem).wait()
            pltpu.sync_copy(rows_v, out_hbm.at[sl, :])
            return carry

        lax.fori_loop(0, nb, b, 0)

    return gat


_R = 256


@functools.lru_cache(maxsize=None)
def _tc_pre(n_pad, d_in, mode):
    grid = (n_pad // _R,)
    deg_spec = pl.BlockSpec((NW, _R, 1), lambda i: (0, i, 0))
    w_spec = pl.BlockSpec((d_in, D), lambda i: (0, 0))
    x2_spec = pl.BlockSpec((_R, d_in), lambda i: (i, 0))
    x3_spec = pl.BlockSpec((2, _R, d_in), lambda i: (0, i, 0))

    if mode == "one":
        in_specs = [x2_spec, deg_spec, w_spec]
    elif mode == "pair":
        in_specs = [x3_spec, deg_spec, w_spec]
    else:
        in_specs = [x2_spec, x2_spec, deg_spec, w_spec]

    def body(*refs):
        if mode == "one":
            x_ref, deg_ref, w_ref, o_ref = refs
            xb = x_ref[...]
        elif mode == "pair":
            x_ref, deg_ref, w_ref, o_ref = refs
            xb = x_ref[0] + x_ref[1]
        else:
            x_ref, s_ref, deg_ref, w_ref, o_ref = refs
            xb = x_ref[...] + s_ref[...]
        deg = jnp.sum(deg_ref[...], axis=0) + 1.0
        dis = lax.rsqrt(deg)
        h = jnp.dot(xb, w_ref[...], preferred_element_type=jnp.float32)
        o_ref[...] = h * dis

    return pl.pallas_call(
        body,
        grid=grid,
        in_specs=in_specs,
        out_specs=pl.BlockSpec((_R, D), lambda i: (i, 0)),
        out_shape=jax.ShapeDtypeStruct((n_pad, D), jnp.float32),
    )


@functools.lru_cache(maxsize=None)
def _tc_post(n_pad, final_rows):
    if final_rows:
        rb = 80
        n_out = final_rows
    else:
        rb = _R
        n_out = n_pad
    grid = (n_out // rb,)

    def body(p_ref, hs_ref, deg_ref, b_ref, o_ref):
        deg = jnp.sum(deg_ref[...], axis=0) + 1.0
        dis = lax.rsqrt(deg)
        t = (p_ref[0] + p_ref[1] + hs_ref[...]) * dis + b_ref[0:1, :]
        t = jnp.maximum(t, 0.0)
        if final_rows:
            nrm = jnp.sqrt(jnp.sum(t * t, axis=1, keepdims=True))
            t = t / jnp.maximum(nrm, 1e-12)
        o_ref[...] = t

    return pl.pallas_call(
        body,
        grid=grid,
        in_specs=[
            pl.BlockSpec((2, rb, D), lambda i: (0, i, 0)),
            pl.BlockSpec((rb, D), lambda i: (i, 0)),
            pl.BlockSpec((NW, rb, 1), lambda i: (0, i, 0)),
            pl.BlockSpec((8, D), lambda i: (0, 0)),
        ],
        out_specs=pl.BlockSpec((rb, D), lambda i: (i, 0)),
        out_shape=jax.ShapeDtypeStruct((n_out, D), jnp.float32),
    )


def _pad_idx(idx, n_pad, fill):
    return jnp.concatenate(
        [idx, jnp.full((n_pad - idx.shape[0],), fill, jnp.int32)]
    )


def kernel(x, edge_index, edge_index_l1, edge_index_l2, cluster0, cluster1,
           W_enc0, b_enc0, W_enc1, b_enc1, W_enc2, b_enc2, W_bot, b_bot,
           W_dec0, b_dec0, W_dec1, b_dec1, W_dec2, b_dec2):
    N0p, N1p, N2p = 10240, 5120, 2560
    E0p, E1p, E2p = 323584, 163840, 81920

    bcast = lambda b: jnp.broadcast_to(b.reshape(1, D), (8, D))

    s0 = _pad_idx(edge_index[0], E0p, N0)
    d0 = _pad_idx(edge_index[1], E0p, N0)
    s1 = _pad_idx(edge_index_l1[0], E1p, N1)
    d1 = _pad_idx(edge_index_l1[1], E1p, N1)
    s2 = _pad_idx(edge_index_l2[0], E2p, N2)
    d2 = _pad_idx(edge_index_l2[1], E2p, N2)
    s2b = _pad_idx(edge_index_l2[0], E2p, N1)
    d2b = _pad_idx(edge_index_l2[1], E2p, N1)
    s1b = _pad_idx(edge_index_l1[0], E1p, N0)
    d1b = _pad_idx(edge_index_l1[1], E1p, N0)
    c0p = _pad_idx(cluster0, N0p, N1)
    c1p = _pad_idx(cluster1, N1p, N2)
    c0g = _pad_idx(cluster0, N0p, 0)
    c1g = _pad_idx(cluster1, N1p, 0)
    xp = jnp.pad(x, ((0, N0p - N0), (0, 0)))

    hist0 = _sc_histogram(N0p, E0p)(d0)
    hist1 = _sc_histogram(N1p, E1p)(d1)
    hist2 = _sc_histogram(N2p, E2p)(d2)
    deg0 = hist0.reshape(NW, N0p, 1)
    deg1 = hist1.reshape(NW, N1p, 1)
    deg2 = hist2.reshape(NW, N2p, 1)
    deg2b = jnp.pad(hist2[:, :N2], ((0, 0), (0, N1p - N2))).reshape(NW, N1p, 1)
    deg1b = jnp.pad(hist1[:, :N1], ((0, 0), (0, N0p - N1))).reshape(NW, N0p, 1)

    hs0 = _tc_pre(N0p, D, "one")(xp, deg0, W_enc0)
    p0 = _sc_propagate(E0p, N0p)(hs0, s0, d0)
    out0 = _tc_post(N0p, 0)(p0, hs0, deg0, bcast(b_enc0))

    q1 = _sc_pool(N0p, N1p)(out0, c0p)
    hs1 = _tc_pre(N1p, D, "pair")(q1, deg1, W_enc1)
    p1 = _sc_propagate(E1p, N1p)(hs1, s1, d1)
    out1 = _tc_post(N1p, 0)(p1, hs1, deg1, bcast(b_enc1))

    q2 = _sc_pool(N1p, N2p)(out1, c1p)
    hs2 = _tc_pre(N2p, D, "pair")(q2, deg2, W_enc2)
    p2 = _sc_propagate(E2p, N2p)(hs2, s2, d2)
    out2 = _tc_post(N2p, 0)(p2, hs2, deg2, bcast(b_enc2))

    hsb = _tc_pre(N2p, D, "one")(out2, deg2, W_bot)
    pb = _sc_propagate(E2p, N2p)(hsb, s2, d2)
    outb = _tc_post(N2p, 0)(pb, hsb, deg2, bcast(b_bot))

    g1 = _sc_gather(N1p)(outb, c1g)
    hsd2 = _tc_pre(N1p, D, "two")(g1, out1, deg2b, W_dec2)
    pd2 = _sc_propagate(E2p, N1p)(hsd2, s2b, d2b)
    outd2 = _tc_post(N1p, 0)(pd2, hsd2, deg2b, bcast(b_dec2))

    g0 = _sc_gather(N0p)(outd2, c0g)
    hsd1 = _tc_pre(N0p, D, "two")(g0, out0, deg1b, W_dec1)
    pd1 = _sc_propagate(E1p, N0p)(hsd1, s1b, d1b)
    outd1 = _tc_post(N0p, 0)(pd1, hsd1, deg1b, bcast(b_dec1))

    hsd0 = _tc_pre(N0p, D, "one")(outd1, deg0, W_dec0)
    pd0 = _sc_propagate(E0p, N0p)(hsd0, s0, d0)
    return _tc_post(N0p, N0)(pd0, hsd0, deg0, bcast(b_dec0))
